# baseline (speedup 1.0000x reference)
import functools

import jax
import jax.numpy as jnp
from jax import lax
from jax.experimental import pallas as pl
from jax.experimental.pallas import tpu as pltpu
from jax.experimental.pallas import tpu_sc as plsc

N_NODES = 10000
D = 128
E = 320000
CHUNK = 128
N_TILES = 16
LANES = 16

CHUNKS_PER_TILE = 160
E_PAD = CHUNKS_PER_TILE * N_TILES * CHUNK
BODIES = CHUNKS_PER_TILE // 2
ROWS_PER_TILE = 624


def _gcn_body(table, gidx, sidx, w, zeros, out_u, out_i,
              gA, sA, wA, gB, sB, wB, rows_v, acc, sem, si):
    c = lax.axis_index("c")
    s = lax.axis_index("s")

    ibase = c * E_PAD + s * CHUNKS_PER_TILE * CHUNK
    wbase = s * CHUNKS_PER_TILE * CHUNK

    def idx_fire(k, g_v, s_v, w_v):
        off = k * CHUNK
        pltpu.async_copy(gidx.at[pl.ds(ibase + off, CHUNK)], g_v, si)
        pltpu.async_copy(sidx.at[pl.ds(ibase + off, CHUNK)], s_v, si)
        pltpu.async_copy(w.at[pl.ds(wbase + off, CHUNK)], w_v, si)

    def idx_drain():
        for _ in range(2):
            pltpu.make_async_copy(gidx.at[pl.ds(ibase, CHUNK)],
                                  gA, si).wait()
        pltpu.make_async_copy(w.at[pl.ds(wbase, CHUNK)], wA, si).wait()

    def scale(w_v):
        def scale_body(g, _):
            w_blk = w_v[pl.ds(g * LANES, LANES)]
            for j in range(LANES):
                wv = w_blk[j]
                e = g * LANES + j
                for d2 in range(D // LANES):
                    rows_v[e, pl.ds(d2 * LANES, LANES)] = (
                        rows_v[e, pl.ds(d2 * LANES, LANES)] * wv)
            return 0

        lax.fori_loop(0, CHUNK // LANES, scale_body, 0)

    idx_fire(0, gA, sA, wA)

    r0 = pl.multiple_of(s * ROWS_PER_TILE, 8)
    n_rows = N_NODES - 15 * ROWS_PER_TILE

    @pl.when(s < N_TILES - 1)
    def _():
        pltpu.sync_copy(zeros.at[pl.ds(r0, ROWS_PER_TILE)],
                        acc.at[pl.ds(r0, ROWS_PER_TILE)])

    @pl.when(s == N_TILES - 1)
    def _():
        pltpu.sync_copy(zeros.at[pl.ds(r0, n_rows)],
                        acc.at[pl.ds(r0, n_rows)])

    idx_drain()
    plsc.subcore_barrier()

    def pair_body(t, carry):
        k0 = 2 * t
        pltpu.async_copy(table.at[gA], rows_v, sem).wait()
        idx_fire(k0 + 1, gB, sB, wB)
        scale(wA)
        idx_drain()
        pltpu.sync_copy(rows_v, acc.at[sA], add=True)

        pltpu.async_copy(table.at[gB], rows_v, sem).wait()

        @pl.when(t + 1 < BODIES)
        def _():
            idx_fire(k0 + 2, gA, sA, wA)

        scale(wB)

        @pl.when(t + 1 < BODIES)
        def _():
            idx_drain()

        pltpu.sync_copy(rows_v, acc.at[sB], add=True)
        return carry

    lax.fori_loop(0, BODIES, pair_body, 0)
    plsc.subcore_barrier()

    @pl.when(jnp.logical_and(c == 0, s < N_TILES - 1))
    def _():
        pltpu.sync_copy(acc.at[pl.ds(r0, ROWS_PER_TILE)],
                        out_u.at[pl.ds(r0, ROWS_PER_TILE)])

    @pl.when(jnp.logical_and(c == 0, s == N_TILES - 1))
    def _():
        pltpu.sync_copy(acc.at[pl.ds(r0, n_rows)],
                        out_u.at[pl.ds(r0, n_rows)])

    @pl.when(jnp.logical_and(c == 1, s < N_TILES - 1))
    def _():
        pltpu.sync_copy(acc.at[pl.ds(r0, ROWS_PER_TILE)],
                        out_i.at[pl.ds(r0, ROWS_PER_TILE)])

    @pl.when(jnp.logical_and(c == 1, s == N_TILES - 1))
    def _():
        pltpu.sync_copy(acc.at[pl.ds(r0, n_rows)],
                        out_i.at[pl.ds(r0, n_rows)])


@jax.jit
def _gcn(table, gidx, sidx, w, zeros):
    mesh = plsc.VectorSubcoreMesh(core_axis_name="c", subcore_axis_name="s")
    f = functools.partial(
        pl.kernel,
        mesh=mesh,
        out_type=(jax.ShapeDtypeStruct((N_NODES, D), jnp.float32),
                  jax.ShapeDtypeStruct((N_NODES, D), jnp.float32)),
        scratch_types=[
            pltpu.VMEM((CHUNK,), jnp.int32),
            pltpu.VMEM((CHUNK,), jnp.int32),
            pltpu.VMEM((CHUNK,), jnp.float32),
            pltpu.VMEM((CHUNK,), jnp.int32),
            pltpu.VMEM((CHUNK,), jnp.int32),
            pltpu.VMEM((CHUNK,), jnp.float32),
            pltpu.VMEM((CHUNK, D), jnp.float32),
            pltpu.VMEM_SHARED((N_NODES, D), jnp.float32),
            pltpu.SemaphoreType.DMA,
            pltpu.SemaphoreType.DMA,
        ],
    )(_gcn_body)
    return f(table, gidx, sidx, w, zeros)


def kernel(user_emb, item_emb, edge_index, edge_weight):
    rows = edge_index[0].astype(jnp.int32)
    cols = edge_index[1].astype(jnp.int32)
    pad = E_PAD - E
    zi = jnp.zeros((pad,), jnp.int32)
    table = jnp.concatenate([item_emb, user_emb], axis=0)
    gidx = jnp.concatenate([cols, zi, rows + N_NODES, zi])
    sidx = jnp.concatenate([rows, zi, cols, zi])
    wf = jnp.concatenate([edge_weight, jnp.zeros((pad,), jnp.float32)])
    zeros = jnp.zeros((N_NODES, D), jnp.float32)
    return _gcn(table, gidx, sidx, wf, zeros)


# speedup vs baseline: 1.7222x; 1.7222x over previous
import functools

import jax
import jax.numpy as jnp
from jax import lax
from jax.experimental import pallas as pl
from jax.experimental.pallas import tpu as pltpu
from jax.experimental.pallas import tpu_sc as plsc

N_NODES = 10000
D = 128
E = 320000
CHUNK = 128
N_CHUNKS = E // CHUNK
N_TILES = 16
ROWS_PER_TILE = 624
LANES = 16


def _gcn_body(table, gidx, sidx, w, zeros, out_u, out_i,
              gidx_v, sidx_v, w_v, rows_v, acc, sem, si):
    c = lax.axis_index("c")
    s = lax.axis_index("s")

    r0 = pl.multiple_of(s * ROWS_PER_TILE, 8)
    n_rows = N_NODES - 15 * ROWS_PER_TILE

    @pl.when(s < N_TILES - 1)
    def _():
        pltpu.sync_copy(zeros.at[pl.ds(r0, ROWS_PER_TILE)],
                        acc.at[pl.ds(r0, ROWS_PER_TILE)])

    @pl.when(s == N_TILES - 1)
    def _():
        pltpu.sync_copy(zeros.at[pl.ds(r0, n_rows)],
                        acc.at[pl.ds(r0, n_rows)])

    plsc.subcore_barrier()

    base = N_CHUNKS // N_TILES
    rem = N_CHUNKS % N_TILES
    n_t = base + jnp.where(s < rem, 1, 0)
    start_t = s * base + jnp.minimum(s, rem)

    def chunk_body(k, carry):
        off = k * CHUNK
        goff = c * E + off
        a = pltpu.async_copy(gidx.at[pl.ds(goff, CHUNK)], gidx_v, si)
        b = pltpu.async_copy(sidx.at[pl.ds(goff, CHUNK)], sidx_v, si)
        d = pltpu.async_copy(w.at[pl.ds(off, CHUNK)], w_v, si)
        a.wait()
        b.wait()
        d.wait()
        pltpu.async_copy(table.at[gidx_v], rows_v, sem).wait()

        def scale_body(g, _):
            w_blk = w_v[pl.ds(g * LANES, LANES)]
            for j in range(LANES):
                wv = w_blk[j]
                e = g * LANES + j
                for d2 in range(D // LANES):
                    rows_v[e, pl.ds(d2 * LANES, LANES)] = (
                        rows_v[e, pl.ds(d2 * LANES, LANES)] * wv)
            return 0

        lax.fori_loop(0, CHUNK // LANES, scale_body, 0)

        pltpu.sync_copy(rows_v, acc.at[sidx_v], add=True)
        return carry

    lax.fori_loop(start_t, start_t + n_t, chunk_body, 0)
    plsc.subcore_barrier()

    @pl.when(jnp.logical_and(c == 0, s < N_TILES - 1))
    def _():
        pltpu.sync_copy(acc.at[pl.ds(r0, ROWS_PER_TILE)],
                        out_u.at[pl.ds(r0, ROWS_PER_TILE)])

    @pl.when(jnp.logical_and(c == 0, s == N_TILES - 1))
    def _():
        pltpu.sync_copy(acc.at[pl.ds(r0, n_rows)],
                        out_u.at[pl.ds(r0, n_rows)])

    @pl.when(jnp.logical_and(c == 1, s < N_TILES - 1))
    def _():
        pltpu.sync_copy(acc.at[pl.ds(r0, ROWS_PER_TILE)],
                        out_i.at[pl.ds(r0, ROWS_PER_TILE)])

    @pl.when(jnp.logical_and(c == 1, s == N_TILES - 1))
    def _():
        pltpu.sync_copy(acc.at[pl.ds(r0, n_rows)],
                        out_i.at[pl.ds(r0, n_rows)])


@jax.jit
def _gcn(table, gidx, sidx, w, zeros):
    mesh = plsc.VectorSubcoreMesh(core_axis_name="c", subcore_axis_name="s")
    f = functools.partial(
        pl.kernel,
        mesh=mesh,
        out_type=(jax.ShapeDtypeStruct((N_NODES, D), jnp.float32),
                  jax.ShapeDtypeStruct((N_NODES, D), jnp.float32)),
        scratch_types=[
            pltpu.VMEM((CHUNK,), jnp.int32),
            pltpu.VMEM((CHUNK,), jnp.int32),
            pltpu.VMEM((CHUNK,), jnp.float32),
            pltpu.VMEM((CHUNK, D), jnp.float32),
            pltpu.VMEM_SHARED((N_NODES, D), jnp.float32),
            pltpu.SemaphoreType.DMA,
            pltpu.SemaphoreType.DMA,
        ],
    )(_gcn_body)
    return f(table, gidx, sidx, w, zeros)


def kernel(user_emb, item_emb, edge_index, edge_weight):
    rows = edge_index[0].astype(jnp.int32)
    cols = edge_index[1].astype(jnp.int32)
    table = jnp.concatenate([item_emb, user_emb], axis=0)
    gidx = jnp.concatenate([cols, rows + N_NODES])
    sidx = jnp.concatenate([rows, cols])
    zeros = jnp.zeros((N_NODES, D), jnp.float32)
    return _gcn(table, gidx, sidx, edge_weight, zeros)
